# baseline (speedup 1.0000x reference)
import functools

import jax
import jax.numpy as jnp
from jax import lax
from jax.experimental import pallas as pl
from jax.experimental.pallas import tpu as pltpu
from jax.experimental.pallas import tpu_sc as plsc

_EPS = 1e-8


def _sqrt16(x):
    xc = jnp.maximum(x, jnp.float32(1e-30))
    i = plsc.bitcast(xc, jnp.int32)
    i = jnp.int32(0x5F3759DF) - (i >> 1)
    y = plsc.bitcast(i, jnp.float32)
    half = jnp.float32(0.5) * xc
    for _ in range(3):
        y = y * (jnp.float32(1.5) - half * y * y)
    return xc * y


@jax.jit
def kernel(user_ids, item_ids, user_table, item_table):
    B = user_ids.shape[0]
    D = user_table.shape[1]
    NC, NS, L = 2, 16, 16
    NW = NC * NS
    b_per_w = B // NW
    CH = 128
    n_chunks = b_per_w // CH
    assert B % (NW * L) == 0 and b_per_w % CH == 0 and CH % L == 0

    mesh = plsc.VectorSubcoreMesh(
        core_axis_name="c", subcore_axis_name="s",
        num_cores=NC, num_subcores=NS)

    @functools.partial(
        pl.kernel,
        out_type=jax.ShapeDtypeStruct((B,), jnp.float32),
        mesh=mesh,
        compiler_params=pltpu.CompilerParams(
            needs_layout_passes=False, use_tc_tiling_on_sc=False),
        scratch_types=[
            pltpu.VMEM((b_per_w,), jnp.int32),
            pltpu.VMEM((b_per_w,), jnp.int32),
            pltpu.VMEM((2, CH, D), jnp.float32),
            pltpu.VMEM((2, CH, D), jnp.float32),
            pltpu.VMEM((b_per_w,), jnp.float32),
            pltpu.SemaphoreType.DMA,
            pltpu.SemaphoreType.DMA,
        ],
    )
    def _cosine_sc(uid_hbm, iid_hbm, ut_hbm, it_hbm, out_hbm,
                   uid_v, iid_v, ubuf, ibuf, out_v, sem0, sem1):
        wid = lax.axis_index("s") * NC + lax.axis_index("c")
        base = wid * b_per_w
        pltpu.sync_copy(uid_hbm.at[pl.ds(base, b_per_w)], uid_v)
        pltpu.sync_copy(iid_hbm.at[pl.ds(base, b_per_w)], iid_v)

        sems = (sem0, sem1)

        def fire(ci, slot):
            s = sems[slot]
            cu = pltpu.async_copy(
                ut_hbm.at[uid_v.at[pl.ds(ci * CH, CH)]], ubuf.at[slot], s)
            cv = pltpu.async_copy(
                it_hbm.at[iid_v.at[pl.ds(ci * CH, CH)]], ibuf.at[slot], s)
            return cu, cv

        lane = lax.iota(jnp.int32, L)

        def compute_chunk(ci, slot):
            ub = ubuf.at[slot]
            ib = ibuf.at[slot]

            def group_body(g, _):
                row = g * L + lane
                zero = jnp.zeros((L,), jnp.float32)
                dot, uu, ii = zero, zero, zero
                for d in range(D):
                    col = jnp.full((L,), d, jnp.int32)
                    u = plsc.load_gather(ub, [row, col])
                    v = plsc.load_gather(ib, [row, col])
                    dot = dot + u * v
                    uu = uu + u * u
                    ii = ii + v * v
                n1 = jnp.maximum(_sqrt16(uu), jnp.float32(_EPS))
                n2 = jnp.maximum(_sqrt16(ii), jnp.float32(_EPS))
                out_v[pl.ds(ci * CH + g * L, L)] = dot / (n1 * n2)
                return 0

            lax.fori_loop(0, CH // L, group_body, 0)

        pend = fire(0, 0)
        for ci in range(n_chunks):
            nxt = fire(ci + 1, (ci + 1) % 2) if ci + 1 < n_chunks else None
            pend[0].wait()
            pend[1].wait()
            compute_chunk(ci, ci % 2)
            pend = nxt

        pltpu.sync_copy(out_v, out_hbm.at[pl.ds(base, b_per_w)])

    return _cosine_sc(user_ids, item_ids, user_table, item_table)


# speedup vs baseline: 1.3180x; 1.3180x over previous
import functools

import jax
import jax.numpy as jnp
from jax import lax
from jax.experimental import pallas as pl
from jax.experimental.pallas import tpu as pltpu
from jax.experimental.pallas import tpu_sc as plsc

_EPS = 1e-8


def _sqrt16(x):
    xc = jnp.maximum(x, jnp.float32(1e-30))
    i = plsc.bitcast(xc, jnp.int32)
    i = jnp.int32(0x5F3759DF) - (i >> 1)
    y = plsc.bitcast(i, jnp.float32)
    half = jnp.float32(0.5) * xc
    for _ in range(3):
        y = y * (jnp.float32(1.5) - half * y * y)
    return xc * y


@jax.jit
def kernel(user_ids, item_ids, user_table, item_table):
    B = user_ids.shape[0]
    D = user_table.shape[1]
    DD = 2 * D
    NC, NS, L = 2, 16, 16
    NW = NC * NS
    b_per_w = B // NW
    CH = 128
    n_chunks = b_per_w // CH
    assert B % (NW * L) == 0 and b_per_w % CH == 0 and CH % L == 0

    mesh = plsc.VectorSubcoreMesh(
        core_axis_name="c", subcore_axis_name="s",
        num_cores=NC, num_subcores=NS)

    @functools.partial(
        pl.kernel,
        out_type=jax.ShapeDtypeStruct((B,), jnp.float32),
        mesh=mesh,
        compiler_params=pltpu.CompilerParams(needs_layout_passes=False),
        scratch_types=[
            pltpu.VMEM((b_per_w,), jnp.int32),
            pltpu.VMEM((b_per_w,), jnp.int32),
            pltpu.VMEM((2, CH, DD), jnp.float32),
            pltpu.VMEM((2, CH, DD), jnp.float32),
            pltpu.VMEM((b_per_w,), jnp.float32),
            pltpu.SemaphoreType.DMA,
            pltpu.SemaphoreType.DMA,
        ],
    )
    def _cosine_sc(uid_hbm, iid_hbm, comb_hbm, out_hbm,
                   uid_v, iid_v, ubuf, ibuf, out_v, sem0, sem1):
        wid = lax.axis_index("s") * NC + lax.axis_index("c")
        base = wid * b_per_w
        pltpu.sync_copy(uid_hbm.at[pl.ds(base, b_per_w)], uid_v)
        pltpu.sync_copy(iid_hbm.at[pl.ds(base, b_per_w)], iid_v)

        sems = (sem0, sem1)

        def fire(ci, slot):
            s = sems[slot]
            cu = pltpu.async_copy(
                comb_hbm.at[uid_v.at[pl.ds(ci * CH, CH)]], ubuf.at[slot], s)
            cv = pltpu.async_copy(
                comb_hbm.at[iid_v.at[pl.ds(ci * CH, CH)]], ibuf.at[slot], s)
            return cu, cv

        lane = lax.iota(jnp.int32, L)

        def compute_chunk(ci, slot):
            ub = ubuf.at[slot]
            ib = ibuf.at[slot]

            def group_body(g, _):
                dots = jnp.zeros((L,), jnp.float32)
                uus = jnp.zeros((L,), jnp.float32)
                iis = jnp.zeros((L,), jnp.float32)
                for r in range(L):
                    dot = jnp.zeros((L,), jnp.float32)
                    uu = jnp.zeros((L,), jnp.float32)
                    ii = jnp.zeros((L,), jnp.float32)
                    for c in range(D // L):
                        u = ub[g * L + r, pl.ds(c * L, L)]
                        v = ib[g * L + r, pl.ds(D + c * L, L)]
                        dot = dot + u * v
                        uu = uu + u * u
                        ii = ii + v * v
                    m = lane == r
                    dots = jnp.where(m, jnp.sum(dot), dots)
                    uus = jnp.where(m, jnp.sum(uu), uus)
                    iis = jnp.where(m, jnp.sum(ii), iis)
                n1 = jnp.maximum(_sqrt16(uus), jnp.float32(_EPS))
                n2 = jnp.maximum(_sqrt16(iis), jnp.float32(_EPS))
                out_v[pl.ds(ci * CH + g * L, L)] = dots / (n1 * n2)
                return 0

            lax.fori_loop(0, CH // L, group_body, 0)

        pend = fire(0, 0)
        for ci in range(n_chunks):
            nxt = fire(ci + 1, (ci + 1) % 2) if ci + 1 < n_chunks else None
            pend[0].wait()
            pend[1].wait()
            compute_chunk(ci, ci % 2)
            pend = nxt

        pltpu.sync_copy(out_v, out_hbm.at[pl.ds(base, b_per_w)])

    combined = jnp.concatenate([user_table, item_table], axis=1)
    return _cosine_sc(user_ids, item_ids, combined)
